# baseline (speedup 1.0000x reference)
import functools

import jax
import jax.numpy as jnp
from jax import lax
from jax.experimental import pallas as pl
from jax.experimental.pallas import tpu as pltpu
from jax.experimental.pallas import tpu_sc as plsc

B = 16384
F = 26
D = 16
V = 1000000
EMB = F * D
EMBP = 512
EPS = 1e-5
H1, H2, H3 = 256, 128, 64

NC = 2
NS = 16
NW = NC * NS
N = B * F
PER_W = N // NW
CHUNK = 1664
NI = PER_W // CHUNK

NROW = B * EMBP // D


NTILE = 7813
VP = NTILE * 128
RP_FULL = NTILE // NW
RP_REM = NTILE % NW


def _make_repack():
    mesh = plsc.VectorSubcoreMesh(core_axis_name="c", subcore_axis_name="s")

    @functools.partial(
        pl.kernel,
        mesh=mesh,
        out_type=jax.ShapeDtypeStruct((VP * D,), jnp.float32),
        scratch_types=[
            pltpu.VMEM((16, 128), jnp.float32),
            pltpu.VMEM((16, 128), jnp.float32),
            pltpu.VMEM((2048,), jnp.float32),
            pltpu.VMEM((2048,), jnp.float32),
            pltpu.SemaphoreType.DMA,
            pltpu.SemaphoreType.DMA,
        ],
        compiler_params=pltpu.CompilerParams(needs_layout_passes=False),
    )
    def repack(tT_hbm, out_hbm, ina, inb, outa, outb, sem_in, sem_out):
        wid = lax.axis_index("s") * NC + lax.axis_index("c")
        ins = (ina, inb)
        outs = (outa, outb)
        lane = lax.iota(jnp.int32, 16)

        def shuffle_block(in16, out_v):
            for rl in range(128):
                cols = jnp.full((16,), rl, jnp.int32)
                vals = plsc.load_gather(in16, [lane, cols])
                out_v[pl.ds(rl * 16, 16)] = vals

        def start_in(t, buf):
            rb = t * NW + wid
            pltpu.async_copy(tT_hbm.at[:, pl.ds(rb * 128, 128)], buf, sem_in)

        def drain_in(buf):
            pltpu.make_async_copy(tT_hbm.at[:, pl.ds(0, 128)], buf,
                                  sem_in).wait()

        def drain_out(buf):
            pltpu.make_async_copy(buf, out_hbm.at[pl.ds(0, 2048)],
                                  sem_out).wait()

        start_in(0, ina)
        start_in(1, inb)

        def body(i, carry):
            for bslot in range(2):
                t = 2 * i + bslot
                buf = ins[bslot]
                obuf = outs[bslot]
                drain_in(buf)

                @pl.when(i > 0)
                def _():
                    drain_out(obuf)

                shuffle_block(buf, obuf)
                rb = t * NW + wid
                pltpu.async_copy(obuf, out_hbm.at[pl.ds(rb * 2048, 2048)],
                                 sem_out)

                @pl.when(i <= RP_FULL // 2 - 2)
                def _():
                    start_in(t + 2, buf)
            return carry

        lax.fori_loop(0, RP_FULL // 2, body, 0)
        drain_out(outa)
        drain_out(outb)

        @pl.when(wid < RP_REM)
        def _tail():
            rb = RP_FULL * NW + wid
            pltpu.sync_copy(tT_hbm.at[:, pl.ds(rb * 128, 128)], ina)
            shuffle_block(ina, outa)
            pltpu.sync_copy(outa, out_hbm.at[pl.ds(rb * 2048, 2048)])

    return repack


_repack = _make_repack()


def _make_gather():
    mesh = plsc.VectorSubcoreMesh(core_axis_name="c", subcore_axis_name="s")

    @functools.partial(
        pl.kernel,
        mesh=mesh,
        out_type=jax.ShapeDtypeStruct((NROW, D), jnp.float32),
        scratch_types=[
            pltpu.VMEM((CHUNK,), jnp.int32),
            pltpu.VMEM((CHUNK,), jnp.int32),
            pltpu.VMEM((CHUNK, D), jnp.float32),
            pltpu.SemaphoreType.DMA,
        ],
        compiler_params=pltpu.CompilerParams(use_tc_tiling_on_sc=False),
    )
    def gather(table_hbm, idx_hbm, dst_hbm, out_hbm, idx_v, dst_v, rows_v,
               sem):
        wid = lax.axis_index("s") * NC + lax.axis_index("c")
        base = wid * PER_W

        def body(i, carry):
            off = base + i * CHUNK
            pltpu.sync_copy(idx_hbm.at[pl.ds(off, CHUNK)], idx_v)
            pltpu.sync_copy(dst_hbm.at[pl.ds(off, CHUNK)], dst_v)
            pltpu.async_copy(table_hbm.at[idx_v], rows_v, sem).wait()
            pltpu.async_copy(rows_v, out_hbm.at[dst_v], sem).wait()
            return carry

        lax.fori_loop(0, NI, body, 0)

    return gather


_gather = _make_gather()


STATS_TB = 2048
STATS_NB = B // STATS_TB


def _stats_kernel(h_ref, gamma_ref, beta_ref, scale_ref, shift_ref,
                  sum_ref, sumsq_ref):
    i = pl.program_id(0)
    x = h_ref[...].reshape(STATS_TB // 8, 4, 8, 128)
    s = jnp.sum(x, axis=(0, 2))
    s2 = jnp.sum(x * x, axis=(0, 2))

    @pl.when(i == 0)
    def _init():
        sum_ref[...] = s
        sumsq_ref[...] = s2

    @pl.when(i > 0)
    def _acc():
        sum_ref[...] += s
        sumsq_ref[...] += s2

    @pl.when(i == STATS_NB - 1)
    def _finish():
        mean = sum_ref[...] * (1.0 / B)
        var = sumsq_ref[...] * (1.0 / B) - mean * mean
        rstd = lax.rsqrt(var + EPS)
        scl = gamma_ref[...] * rstd
        scale_ref[...] = scl
        shift_ref[...] = beta_ref[...] - mean * scl


def _stats(h_lin, gamma4, beta4):
    return pl.pallas_call(
        _stats_kernel,
        grid=(STATS_NB,),
        in_specs=[
            pl.BlockSpec((STATS_TB * 4, 128), lambda i: (i, 0)),
            pl.BlockSpec((4, 128), lambda i: (0, 0)),
            pl.BlockSpec((4, 128), lambda i: (0, 0)),
        ],
        out_specs=[
            pl.BlockSpec((4, 128), lambda i: (0, 0)),
            pl.BlockSpec((4, 128), lambda i: (0, 0)),
        ],
        out_shape=[
            jax.ShapeDtypeStruct((4, 128), jnp.float32),
            jax.ShapeDtypeStruct((4, 128), jnp.float32),
        ],
        scratch_shapes=[
            pltpu.VMEM((4, 128), jnp.float32),
            pltpu.VMEM((4, 128), jnp.float32),
        ],
    )(h_lin, gamma4, beta4)


MLP_TB = 1024
MLP_NB = B // MLP_TB


def _mlp_kernel(h_ref, scale_ref, shift_ref, W1_ref, b1_ref, W2_ref, b2_ref,
                W3_ref, b3_ref, out_ref):
    x = h_ref[...].reshape(MLP_TB // 8, 4, 8, 128)
    y = x * scale_ref[...][None, :, None, :] + shift_ref[...][None, :, None, :]
    a = jnp.dot(y[:, 0].reshape(MLP_TB, 128), W1_ref[0:128],
                preferred_element_type=jnp.float32)
    a += jnp.dot(y[:, 1].reshape(MLP_TB, 128), W1_ref[128:256],
                 preferred_element_type=jnp.float32)
    a += jnp.dot(y[:, 2].reshape(MLP_TB, 128), W1_ref[256:384],
                 preferred_element_type=jnp.float32)
    a += jnp.dot(y[:, 3].reshape(MLP_TB, 128)[:, 0:32], W1_ref[384:416],
                 preferred_element_type=jnp.float32)
    a = jnp.maximum(a + b1_ref[...], 0.0)
    a = jnp.maximum(jnp.dot(a, W2_ref[...],
                            preferred_element_type=jnp.float32)
                    + b2_ref[...], 0.0)
    out_ref[...] = jnp.dot(a, W3_ref[...],
                           preferred_element_type=jnp.float32) + b3_ref[...]


def _mlp(h_lin, scale, shift, W1, b1, W2, b2, W3, b3):
    return pl.pallas_call(
        _mlp_kernel,
        grid=(MLP_NB,),
        in_specs=[
            pl.BlockSpec((MLP_TB * 4, 128), lambda i: (i, 0)),
            pl.BlockSpec((4, 128), lambda i: (0, 0)),
            pl.BlockSpec((4, 128), lambda i: (0, 0)),
            pl.BlockSpec((EMB, H1), lambda i: (0, 0)),
            pl.BlockSpec((1, H1), lambda i: (0, 0)),
            pl.BlockSpec((H1, H2), lambda i: (0, 0)),
            pl.BlockSpec((1, H2), lambda i: (0, 0)),
            pl.BlockSpec((H2, H3), lambda i: (0, 0)),
            pl.BlockSpec((1, H3), lambda i: (0, 0)),
        ],
        out_specs=pl.BlockSpec((MLP_TB, H3), lambda i: (i, 0)),
        out_shape=jax.ShapeDtypeStruct((B, H3), jnp.float32),
    )(h_lin, scale, shift, W1, b1.reshape(1, H1), W2, b2.reshape(1, H2),
      W3, b3.reshape(1, H3))


@jax.jit
def kernel(x, table, gamma, beta, W1, b1, W2, b2, W3, b3):
    table_lin = _repack(table.T).reshape(VP, D)
    flat_idx = x.reshape(N)
    j = jnp.arange(N, dtype=jnp.int32)
    b_i = j // F
    f_i = j % F
    dst = ((b_i >> 3) * 4 + (f_i >> 3)) * 64 + (b_i & 7) * 8 + (f_i & 7)
    h_flat = _gather(table_lin, flat_idx, dst)
    h_lin = h_flat.reshape(B * EMBP // 128, 128)
    gamma4 = jnp.pad(gamma, (0, EMBP - EMB)).reshape(4, 128)
    beta4 = jnp.pad(beta, (0, EMBP - EMB)).reshape(4, 128)
    scale, shift = _stats(h_lin, gamma4, beta4)
    return _mlp(h_lin, scale, shift, W1, b1, W2, b2, W3, b3)


# speedup vs baseline: 2.3328x; 2.3328x over previous
import functools

import jax
import jax.numpy as jnp
from jax import lax
from jax.experimental import pallas as pl
from jax.experimental.pallas import tpu as pltpu
from jax.experimental.pallas import tpu_sc as plsc

B = 16384
F = 26
D = 16
V = 1000000
EMB = F * D
EMBP = 512
EPS = 1e-5
H1, H2, H3 = 256, 128, 64

NC = 2
NS = 16
NW = NC * NS
N = B * F
PER_W = N // NW
CHUNK = 1664
NI = PER_W // CHUNK

NROW = B * EMBP // D


NTILE = 7813
VP = NTILE * 128
RP_FULL = NTILE // NW
RP_REM = NTILE % NW


def _make_repack():
    mesh = plsc.VectorSubcoreMesh(core_axis_name="c", subcore_axis_name="s")

    @functools.partial(
        pl.kernel,
        mesh=mesh,
        out_type=jax.ShapeDtypeStruct((VP * D,), jnp.float32),
        scratch_types=[
            pltpu.VMEM((16, 128), jnp.float32),
            pltpu.VMEM((16, 128), jnp.float32),
            pltpu.VMEM((2048,), jnp.float32),
            pltpu.VMEM((2048,), jnp.float32),
            pltpu.SemaphoreType.DMA,
            pltpu.SemaphoreType.DMA,
        ],
        compiler_params=pltpu.CompilerParams(needs_layout_passes=False),
    )
    def repack(tT_hbm, out_hbm, ina, inb, outa, outb, sem_in, sem_out):
        wid = lax.axis_index("s") * NC + lax.axis_index("c")
        ins = (ina, inb)
        outs = (outa, outb)
        lane = lax.iota(jnp.int32, 16)

        def shuffle_block(in16, out_v):
            base16 = lane * 16
            @plsc.parallel_loop(0, 16, unroll=4)
            def _body(c):
                for k in range(8):
                    vals = in16[c, pl.ds(k * 16, 16)]
                    idx = base16 + (256 * k + c)
                    plsc.store_scatter(out_v, [idx], vals)

        def start_in(t, buf):
            rb = t * NW + wid
            pltpu.async_copy(tT_hbm.at[:, pl.ds(rb * 128, 128)], buf, sem_in)

        def drain_in(buf):
            pltpu.make_async_copy(tT_hbm.at[:, pl.ds(0, 128)], buf,
                                  sem_in).wait()

        def drain_out(buf):
            pltpu.make_async_copy(buf, out_hbm.at[pl.ds(0, 2048)],
                                  sem_out).wait()

        start_in(0, ina)
        start_in(1, inb)

        def body(i, carry):
            for bslot in range(2):
                t = 2 * i + bslot
                buf = ins[bslot]
                obuf = outs[bslot]
                drain_in(buf)

                @pl.when(i > 0)
                def _():
                    drain_out(obuf)

                shuffle_block(buf, obuf)
                rb = t * NW + wid
                pltpu.async_copy(obuf, out_hbm.at[pl.ds(rb * 2048, 2048)],
                                 sem_out)

                @pl.when(i <= RP_FULL // 2 - 2)
                def _():
                    start_in(t + 2, buf)
            return carry

        lax.fori_loop(0, RP_FULL // 2, body, 0)
        drain_out(outa)
        drain_out(outb)

        @pl.when(wid < RP_REM)
        def _tail():
            rb = RP_FULL * NW + wid
            pltpu.sync_copy(tT_hbm.at[:, pl.ds(rb * 128, 128)], ina)
            shuffle_block(ina, outa)
            pltpu.sync_copy(outa, out_hbm.at[pl.ds(rb * 2048, 2048)])

    return repack


_repack = _make_repack()


def _make_gather():
    mesh = plsc.VectorSubcoreMesh(core_axis_name="c", subcore_axis_name="s")

    @functools.partial(
        pl.kernel,
        mesh=mesh,
        out_type=jax.ShapeDtypeStruct((NROW, D), jnp.float32),
        scratch_types=[
            pltpu.VMEM((CHUNK,), jnp.int32),
            pltpu.VMEM((CHUNK,), jnp.int32),
            pltpu.VMEM((CHUNK, D), jnp.float32),
            pltpu.SemaphoreType.DMA,
        ],
        compiler_params=pltpu.CompilerParams(use_tc_tiling_on_sc=False),
    )
    def gather(table_hbm, idx_hbm, dst_hbm, out_hbm, idx_v, dst_v, rows_v,
               sem):
        wid = lax.axis_index("s") * NC + lax.axis_index("c")
        base = wid * PER_W

        def body(i, carry):
            off = base + i * CHUNK
            pltpu.sync_copy(idx_hbm.at[pl.ds(off, CHUNK)], idx_v)
            pltpu.sync_copy(dst_hbm.at[pl.ds(off, CHUNK)], dst_v)
            pltpu.async_copy(table_hbm.at[idx_v], rows_v, sem).wait()
            pltpu.async_copy(rows_v, out_hbm.at[dst_v], sem).wait()
            return carry

        lax.fori_loop(0, NI, body, 0)

    return gather


_gather = _make_gather()


STATS_TB = 2048
STATS_NB = B // STATS_TB


def _stats_kernel(h_ref, gamma_ref, beta_ref, scale_ref, shift_ref,
                  sum_ref, sumsq_ref):
    i = pl.program_id(0)
    x = h_ref[...].reshape(STATS_TB // 8, 4, 8, 128)
    s = jnp.sum(x, axis=(0, 2))
    s2 = jnp.sum(x * x, axis=(0, 2))

    @pl.when(i == 0)
    def _init():
        sum_ref[...] = s
        sumsq_ref[...] = s2

    @pl.when(i > 0)
    def _acc():
        sum_ref[...] += s
        sumsq_ref[...] += s2

    @pl.when(i == STATS_NB - 1)
    def _finish():
        mean = sum_ref[...] * (1.0 / B)
        var = sumsq_ref[...] * (1.0 / B) - mean * mean
        rstd = lax.rsqrt(var + EPS)
        scl = gamma_ref[...] * rstd
        scale_ref[...] = scl
        shift_ref[...] = beta_ref[...] - mean * scl


def _stats(h_lin, gamma4, beta4):
    return pl.pallas_call(
        _stats_kernel,
        grid=(STATS_NB,),
        in_specs=[
            pl.BlockSpec((STATS_TB * 4, 128), lambda i: (i, 0)),
            pl.BlockSpec((4, 128), lambda i: (0, 0)),
            pl.BlockSpec((4, 128), lambda i: (0, 0)),
        ],
        out_specs=[
            pl.BlockSpec((4, 128), lambda i: (0, 0)),
            pl.BlockSpec((4, 128), lambda i: (0, 0)),
        ],
        out_shape=[
            jax.ShapeDtypeStruct((4, 128), jnp.float32),
            jax.ShapeDtypeStruct((4, 128), jnp.float32),
        ],
        scratch_shapes=[
            pltpu.VMEM((4, 128), jnp.float32),
            pltpu.VMEM((4, 128), jnp.float32),
        ],
    )(h_lin, gamma4, beta4)


MLP_TB = 1024
MLP_NB = B // MLP_TB


def _mlp_kernel(h_ref, scale_ref, shift_ref, W1_ref, b1_ref, W2_ref, b2_ref,
                W3_ref, b3_ref, out_ref):
    x = h_ref[...].reshape(MLP_TB // 8, 4, 8, 128)
    y = x * scale_ref[...][None, :, None, :] + shift_ref[...][None, :, None, :]
    a = jnp.dot(y[:, 0].reshape(MLP_TB, 128), W1_ref[0:128],
                preferred_element_type=jnp.float32)
    a += jnp.dot(y[:, 1].reshape(MLP_TB, 128), W1_ref[128:256],
                 preferred_element_type=jnp.float32)
    a += jnp.dot(y[:, 2].reshape(MLP_TB, 128), W1_ref[256:384],
                 preferred_element_type=jnp.float32)
    a += jnp.dot(y[:, 3].reshape(MLP_TB, 128)[:, 0:32], W1_ref[384:416],
                 preferred_element_type=jnp.float32)
    a = jnp.maximum(a + b1_ref[...], 0.0)
    a = jnp.maximum(jnp.dot(a, W2_ref[...],
                            preferred_element_type=jnp.float32)
                    + b2_ref[...], 0.0)
    out_ref[...] = jnp.dot(a, W3_ref[...],
                           preferred_element_type=jnp.float32) + b3_ref[...]


def _mlp(h_lin, scale, shift, W1, b1, W2, b2, W3, b3):
    return pl.pallas_call(
        _mlp_kernel,
        grid=(MLP_NB,),
        in_specs=[
            pl.BlockSpec((MLP_TB * 4, 128), lambda i: (i, 0)),
            pl.BlockSpec((4, 128), lambda i: (0, 0)),
            pl.BlockSpec((4, 128), lambda i: (0, 0)),
            pl.BlockSpec((EMB, H1), lambda i: (0, 0)),
            pl.BlockSpec((1, H1), lambda i: (0, 0)),
            pl.BlockSpec((H1, H2), lambda i: (0, 0)),
            pl.BlockSpec((1, H2), lambda i: (0, 0)),
            pl.BlockSpec((H2, H3), lambda i: (0, 0)),
            pl.BlockSpec((1, H3), lambda i: (0, 0)),
        ],
        out_specs=pl.BlockSpec((MLP_TB, H3), lambda i: (i, 0)),
        out_shape=jax.ShapeDtypeStruct((B, H3), jnp.float32),
    )(h_lin, scale, shift, W1, b1.reshape(1, H1), W2, b2.reshape(1, H2),
      W3, b3.reshape(1, H3))


@jax.jit
def kernel(x, table, gamma, beta, W1, b1, W2, b2, W3, b3):
    table_lin = _repack(table.T).reshape(VP, D)
    flat_idx = x.reshape(N)
    j = jnp.arange(N, dtype=jnp.int32)
    b_i = j // F
    f_i = j % F
    dst = ((b_i >> 3) * 4 + (f_i >> 3)) * 64 + (b_i & 7) * 8 + (f_i & 7)
    h_flat = _gather(table_lin, flat_idx, dst)
    h_lin = h_flat.reshape(B * EMBP // 128, 128)
    gamma4 = jnp.pad(gamma, (0, EMBP - EMB)).reshape(4, 128)
    beta4 = jnp.pad(beta, (0, EMBP - EMB)).reshape(4, 128)
    scale, shift = _stats(h_lin, gamma4, beta4)
    return _mlp(h_lin, scale, shift, W1, b1, W2, b2, W3, b3)


# speedup vs baseline: 2.5063x; 1.0744x over previous
import functools

import jax
import jax.numpy as jnp
from jax import lax
from jax.experimental import pallas as pl
from jax.experimental.pallas import tpu as pltpu
from jax.experimental.pallas import tpu_sc as plsc

B = 16384
F = 26
D = 16
V = 1000000
EMB = F * D
EMBP = 512
EPS = 1e-5
H1, H2, H3 = 256, 128, 64

NC = 2
NS = 16
NW = NC * NS
N = B * F
PER_W = N // NW
CHUNK = 1664
NI = PER_W // CHUNK

NROW = B * EMBP // D


NTILE = 7813
VP = NTILE * 128
RP_FULL = NTILE // NW
RP_REM = NTILE % NW


def _make_repack():
    mesh = plsc.VectorSubcoreMesh(core_axis_name="c", subcore_axis_name="s")

    @functools.partial(
        pl.kernel,
        mesh=mesh,
        out_type=jax.ShapeDtypeStruct((VP * D,), jnp.float32),
        scratch_types=[
            pltpu.VMEM((16, 128), jnp.float32),
            pltpu.VMEM((16, 128), jnp.float32),
            pltpu.VMEM((2048,), jnp.float32),
            pltpu.VMEM((2048,), jnp.float32),
            pltpu.SemaphoreType.DMA,
            pltpu.SemaphoreType.DMA,
        ],
        compiler_params=pltpu.CompilerParams(needs_layout_passes=False),
    )
    def repack(tT_hbm, out_hbm, ina, inb, outa, outb, sem_in, sem_out):
        wid = lax.axis_index("s") * NC + lax.axis_index("c")
        ins = (ina, inb)
        outs = (outa, outb)
        lane = lax.iota(jnp.int32, 16)

        def shuffle_block(in16, out_v):
            base16 = lane * 16
            @plsc.parallel_loop(0, 16, unroll=8)
            def _body(c):
                for k in range(8):
                    vals = in16[c, pl.ds(k * 16, 16)]
                    idx = base16 + (256 * k + c)
                    plsc.store_scatter(out_v, [idx], vals)

        def start_in(t, buf):
            rb = t * NW + wid
            pltpu.async_copy(tT_hbm.at[:, pl.ds(rb * 128, 128)], buf, sem_in)

        def drain_in(buf):
            pltpu.make_async_copy(tT_hbm.at[:, pl.ds(0, 128)], buf,
                                  sem_in).wait()

        def drain_out(buf):
            pltpu.make_async_copy(buf, out_hbm.at[pl.ds(0, 2048)],
                                  sem_out).wait()

        start_in(0, ina)
        start_in(1, inb)

        def body(i, carry):
            for bslot in range(2):
                t = 2 * i + bslot
                buf = ins[bslot]
                obuf = outs[bslot]
                drain_in(buf)

                @pl.when(i > 0)
                def _():
                    drain_out(obuf)

                shuffle_block(buf, obuf)
                rb = t * NW + wid
                pltpu.async_copy(obuf, out_hbm.at[pl.ds(rb * 2048, 2048)],
                                 sem_out)

                @pl.when(i <= RP_FULL // 2 - 2)
                def _():
                    start_in(t + 2, buf)
            return carry

        lax.fori_loop(0, RP_FULL // 2, body, 0)
        drain_out(outa)
        drain_out(outb)

        @pl.when(wid < RP_REM)
        def _tail():
            rb = RP_FULL * NW + wid
            pltpu.sync_copy(tT_hbm.at[:, pl.ds(rb * 128, 128)], ina)
            shuffle_block(ina, outa)
            pltpu.sync_copy(outa, out_hbm.at[pl.ds(rb * 2048, 2048)])

    return repack


_repack = _make_repack()


def _make_gather():
    mesh = plsc.VectorSubcoreMesh(core_axis_name="c", subcore_axis_name="s")

    @functools.partial(
        pl.kernel,
        mesh=mesh,
        out_type=jax.ShapeDtypeStruct((NROW, D), jnp.float32),
        scratch_types=[
            pltpu.VMEM((CHUNK,), jnp.int32),
            pltpu.VMEM((CHUNK,), jnp.int32),
            pltpu.VMEM((CHUNK,), jnp.int32),
            pltpu.VMEM((CHUNK,), jnp.int32),
            pltpu.VMEM((CHUNK, D), jnp.float32),
            pltpu.VMEM((CHUNK, D), jnp.float32),
            pltpu.SemaphoreType.DMA((2,)),
            pltpu.SemaphoreType.DMA((2,)),
            pltpu.SemaphoreType.DMA((2,)),
        ],
        compiler_params=pltpu.CompilerParams(use_tc_tiling_on_sc=False),
    )
    def gather(table_hbm, idx_hbm, dst_hbm, out_hbm,
               idx_a, idx_b, dst_a, dst_b, rows_a, rows_b,
               sem_i, sem_g, sem_s):
        wid = lax.axis_index("s") * NC + lax.axis_index("c")
        base = wid * PER_W
        idxs = (idx_a, idx_b)
        dsts = (dst_a, dst_b)
        rows = (rows_a, rows_b)

        def start_idx(t):
            off = base + t * CHUNK
            pltpu.async_copy(idx_hbm.at[pl.ds(off, CHUNK)], idxs[t % 2],
                             sem_i.at[t % 2])
            pltpu.async_copy(dst_hbm.at[pl.ds(off, CHUNK)], dsts[t % 2],
                             sem_i.at[t % 2])

        def wait_idx(t):
            pltpu.make_async_copy(idx_hbm.at[pl.ds(0, CHUNK)], idxs[t % 2],
                                  sem_i.at[t % 2]).wait()
            pltpu.make_async_copy(idx_hbm.at[pl.ds(0, CHUNK)], dsts[t % 2],
                                  sem_i.at[t % 2]).wait()

        def start_gather(t):
            pltpu.async_copy(table_hbm.at[idxs[t % 2]], rows[t % 2],
                             sem_g.at[t % 2])

        def wait_gather(t):
            pltpu.make_async_copy(table_hbm.at[pl.ds(0, CHUNK)], rows[t % 2],
                                  sem_g.at[t % 2]).wait()

        def start_scatter(t):
            pltpu.async_copy(rows[t % 2], out_hbm.at[dsts[t % 2]],
                             sem_s.at[t % 2])

        def wait_scatter(t):
            pltpu.make_async_copy(rows[t % 2], out_hbm.at[pl.ds(0, CHUNK)],
                                  sem_s.at[t % 2]).wait()

        start_idx(0)
        start_idx(1)
        wait_idx(0)
        start_gather(0)
        for t in range(NI):
            if t >= 1:
                wait_scatter(t - 1)
            if t + 1 < NI:
                wait_idx(t + 1)
                start_gather(t + 1)
            wait_gather(t)
            start_scatter(t)
            if t + 2 < NI:
                start_idx(t + 2)
        wait_scatter(NI - 1)

    return gather


_gather = _make_gather()


STATS_TB = 2048
STATS_NB = B // STATS_TB


def _stats_kernel(h_ref, gamma_ref, beta_ref, scale_ref, shift_ref,
                  sum_ref, sumsq_ref):
    i = pl.program_id(0)
    x = h_ref[...].reshape(STATS_TB // 8, 4, 8, 128)
    s = jnp.sum(x, axis=(0, 2))
    s2 = jnp.sum(x * x, axis=(0, 2))

    @pl.when(i == 0)
    def _init():
        sum_ref[...] = s
        sumsq_ref[...] = s2

    @pl.when(i > 0)
    def _acc():
        sum_ref[...] += s
        sumsq_ref[...] += s2

    @pl.when(i == STATS_NB - 1)
    def _finish():
        mean = sum_ref[...] * (1.0 / B)
        var = sumsq_ref[...] * (1.0 / B) - mean * mean
        rstd = lax.rsqrt(var + EPS)
        scl = gamma_ref[...] * rstd
        scale_ref[...] = scl
        shift_ref[...] = beta_ref[...] - mean * scl


def _stats(h_lin, gamma4, beta4):
    return pl.pallas_call(
        _stats_kernel,
        grid=(STATS_NB,),
        in_specs=[
            pl.BlockSpec((STATS_TB * 4, 128), lambda i: (i, 0)),
            pl.BlockSpec((4, 128), lambda i: (0, 0)),
            pl.BlockSpec((4, 128), lambda i: (0, 0)),
        ],
        out_specs=[
            pl.BlockSpec((4, 128), lambda i: (0, 0)),
            pl.BlockSpec((4, 128), lambda i: (0, 0)),
        ],
        out_shape=[
            jax.ShapeDtypeStruct((4, 128), jnp.float32),
            jax.ShapeDtypeStruct((4, 128), jnp.float32),
        ],
        scratch_shapes=[
            pltpu.VMEM((4, 128), jnp.float32),
            pltpu.VMEM((4, 128), jnp.float32),
        ],
    )(h_lin, gamma4, beta4)


MLP_TB = 1024
MLP_NB = B // MLP_TB


def _mlp_kernel(h_ref, scale_ref, shift_ref, W1_ref, b1_ref, W2_ref, b2_ref,
                W3_ref, b3_ref, out_ref):
    x = h_ref[...].reshape(MLP_TB // 8, 4, 8, 128)
    y = x * scale_ref[...][None, :, None, :] + shift_ref[...][None, :, None, :]
    a = jnp.dot(y[:, 0].reshape(MLP_TB, 128), W1_ref[0:128],
                preferred_element_type=jnp.float32)
    a += jnp.dot(y[:, 1].reshape(MLP_TB, 128), W1_ref[128:256],
                 preferred_element_type=jnp.float32)
    a += jnp.dot(y[:, 2].reshape(MLP_TB, 128), W1_ref[256:384],
                 preferred_element_type=jnp.float32)
    a += jnp.dot(y[:, 3].reshape(MLP_TB, 128)[:, 0:32], W1_ref[384:416],
                 preferred_element_type=jnp.float32)
    a = jnp.maximum(a + b1_ref[...], 0.0)
    a = jnp.maximum(jnp.dot(a, W2_ref[...],
                            preferred_element_type=jnp.float32)
                    + b2_ref[...], 0.0)
    out_ref[...] = jnp.dot(a, W3_ref[...],
                           preferred_element_type=jnp.float32) + b3_ref[...]


def _mlp(h_lin, scale, shift, W1, b1, W2, b2, W3, b3):
    return pl.pallas_call(
        _mlp_kernel,
        grid=(MLP_NB,),
        in_specs=[
            pl.BlockSpec((MLP_TB * 4, 128), lambda i: (i, 0)),
            pl.BlockSpec((4, 128), lambda i: (0, 0)),
            pl.BlockSpec((4, 128), lambda i: (0, 0)),
            pl.BlockSpec((EMB, H1), lambda i: (0, 0)),
            pl.BlockSpec((1, H1), lambda i: (0, 0)),
            pl.BlockSpec((H1, H2), lambda i: (0, 0)),
            pl.BlockSpec((1, H2), lambda i: (0, 0)),
            pl.BlockSpec((H2, H3), lambda i: (0, 0)),
            pl.BlockSpec((1, H3), lambda i: (0, 0)),
        ],
        out_specs=pl.BlockSpec((MLP_TB, H3), lambda i: (i, 0)),
        out_shape=jax.ShapeDtypeStruct((B, H3), jnp.float32),
    )(h_lin, scale, shift, W1, b1.reshape(1, H1), W2, b2.reshape(1, H2),
      W3, b3.reshape(1, H3))


@jax.jit
def kernel(x, table, gamma, beta, W1, b1, W2, b2, W3, b3):
    table_lin = _repack(table.T).reshape(VP, D)
    flat_idx = x.reshape(N)
    j = jnp.arange(N, dtype=jnp.int32)
    b_i = j // F
    f_i = j % F
    dst = ((b_i >> 3) * 4 + (f_i >> 3)) * 64 + (b_i & 7) * 8 + (f_i & 7)
    h_flat = _gather(table_lin, flat_idx, dst)
    h_lin = h_flat.reshape(B * EMBP // 128, 128)
    gamma4 = jnp.pad(gamma, (0, EMBP - EMB)).reshape(4, 128)
    beta4 = jnp.pad(beta, (0, EMBP - EMB)).reshape(4, 128)
    scale, shift = _stats(h_lin, gamma4, beta4)
    return _mlp(h_lin, scale, shift, W1, b1, W2, b2, W3, b3)
